# baseline (speedup 1.0000x reference)
import dataclasses
import functools

import jax
import jax.numpy as jnp
from jax.experimental import pallas as pl
from jax.experimental.pallas import tpu as pltpu
from jax.experimental.pallas import tpu_sc as plsc

_N = 1_000_000
_B_SC = 2_000
_G_SC = _B_SC // 16
_OBS_VAL = 1000.0


def _sc_indicator(xt, n):
    mesh = plsc.VectorSubcoreMesh(core_axis_name="c", subcore_axis_name="s")
    cp = pltpu.CompilerParams()
    if "needs_layout_passes" in pltpu.CompilerParams.__dataclass_fields__:
        cp = dataclasses.replace(cp, needs_layout_passes=False)
    if "use_tc_tiling_on_sc" in pltpu.CompilerParams.__dataclass_fields__:
        cp = dataclasses.replace(cp, use_tc_tiling_on_sc=False)

    @functools.partial(
        pl.kernel,
        out_type=jax.ShapeDtypeStruct((n,), jnp.float32),
        mesh=mesh,
        compiler_params=cp,
    )
    def sc_kernel(x_hbm, o_hbm):
        def body(x_vmem, o_vmem):
            @plsc.parallel_loop(0, _G_SC, 1, unroll=4)
            def _(g):
                sl = pl.ds(g * 16, 16)
                e = x_vmem[0, sl]
                o = x_vmem[1, sl]
                m = (jnp.abs(e) <= 3.0) & (jnp.abs(o) <= 1.5)
                o_vmem[sl] = jnp.where(m, _OBS_VAL, 0.0).astype(jnp.float32)

        pltpu.emit_pipeline(
            body,
            grid=(n // _B_SC,),
            in_specs=[pl.BlockSpec((2, _B_SC), lambda i: (0, i))],
            out_specs=[pl.BlockSpec((_B_SC,), lambda i: (i,))],
            core_axis_name=("c", "s"),
            dimension_semantics=(pltpu.PARALLEL,),
        )(x_hbm, o_hbm)

    return sc_kernel(xt)


def kernel(x):
    out = _sc_indicator(x.T, _N)
    return out.reshape(_N, 1)


# speedup vs baseline: 8.6885x; 8.6885x over previous
import jax
import jax.numpy as jnp
from jax.experimental import pallas as pl

_N = 1_000_000
_OBS_VAL = 1000.0


def _tc_indicator(xt):
    def body(x_ref, o_ref):
        e = x_ref[0:1, :]
        o = x_ref[1:2, :]
        m = (jnp.abs(e) <= 3.0) & (jnp.abs(o) <= 1.5)
        o_ref[...] = jnp.where(m, jnp.float32(_OBS_VAL), jnp.float32(0.0))

    return pl.pallas_call(
        body,
        out_shape=jax.ShapeDtypeStruct((1, _N), jnp.float32),
    )(xt)


def kernel(x):
    out = _tc_indicator(x.T)
    return out.reshape(_N, 1)
